# baseline (speedup 1.0000x reference)
import functools

import jax
import jax.numpy as jnp
from jax import lax
from jax.experimental import pallas as pl
from jax.experimental.pallas import tpu as pltpu
from jax.experimental.pallas import tpu_sc as plsc


@functools.lru_cache(maxsize=None)
def _make_gather(V, D, B, NCH):
    info = plsc.get_sparse_core_info()
    NC, NS = info.num_cores, info.num_subcores
    NW = NC * NS
    assert B % (8 * NW) == 0
    b_per_w = B // NW
    assert b_per_w % NCH == 0
    C = b_per_w // NCH
    mesh = plsc.VectorSubcoreMesh(core_axis_name="c", subcore_axis_name="s")

    @functools.partial(
        pl.kernel,
        mesh=mesh,
        out_type=jax.ShapeDtypeStruct((B, D), jnp.float32),
        scratch_types=[
            pltpu.VMEM((NCH, C), jnp.int32),
            pltpu.VMEM((NCH, C, D), jnp.float32),
            pltpu.SemaphoreType.DMA((NCH,)),
            pltpu.SemaphoreType.DMA,
        ],
        compiler_params=pltpu.CompilerParams(use_tc_tiling_on_sc=False),
    )
    def gather_kernel(table_hbm, idx_hbm, out_hbm, idx_v, rows_v, gsem, wsem):
        wid = lax.axis_index("s") * NC + lax.axis_index("c")
        base = wid * b_per_w
        pltpu.sync_copy(idx_hbm.at[wid], idx_v)
        gathers = [
            pltpu.async_copy(table_hbm.at[idx_v.at[c]], rows_v.at[c], gsem.at[c])
            for c in range(NCH)
        ]
        writes = []
        for c in range(NCH):
            gathers[c].wait()
            writes.append(
                pltpu.async_copy(
                    rows_v.at[c], out_hbm.at[pl.ds(base + c * C, C)], wsem
                )
            )
        for w in writes:
            w.wait()

    return gather_kernel


@jax.jit
def kernel(label_ids, table):
    B = label_ids.shape[0]
    V, D = table.shape
    NCH = 4
    info = plsc.get_sparse_core_info()
    NW = info.num_cores * info.num_subcores
    idx = label_ids.astype(jnp.int32).reshape(NW, NCH, B // (NW * NCH))
    out = _make_gather(V, D, B, NCH)(table, idx)
    return out[:, None, :]


# speedup vs baseline: 1.0039x; 1.0039x over previous
import functools

import jax
import jax.numpy as jnp
from jax import lax
from jax.experimental import pallas as pl
from jax.experimental.pallas import tpu as pltpu
from jax.experimental.pallas import tpu_sc as plsc


@functools.lru_cache(maxsize=None)
def _make_gather(V, D, B, NCH):
    info = plsc.get_sparse_core_info()
    NC, NS = info.num_cores, info.num_subcores
    NW = NC * NS
    assert B % (8 * NW) == 0
    b_per_w = B // NW
    assert b_per_w % NCH == 0
    C = b_per_w // NCH
    mesh = plsc.VectorSubcoreMesh(core_axis_name="c", subcore_axis_name="s")

    @functools.partial(
        pl.kernel,
        mesh=mesh,
        out_type=jax.ShapeDtypeStruct((B, D), jnp.float32),
        scratch_types=[
            pltpu.VMEM((b_per_w,), jnp.int32),
            pltpu.VMEM((NCH, C, D), jnp.float32),
            pltpu.SemaphoreType.DMA((NCH,)),
            pltpu.SemaphoreType.DMA,
        ],
        compiler_params=pltpu.CompilerParams(use_tc_tiling_on_sc=False),
    )
    def gather_kernel(table_hbm, idx_hbm, out_hbm, idx_v, rows_v, gsem, wsem):
        wid = lax.axis_index("s") * NC + lax.axis_index("c")
        base = wid * b_per_w
        pltpu.sync_copy(idx_hbm.at[pl.ds(base, b_per_w)], idx_v)
        gathers = [
            pltpu.async_copy(
                table_hbm.at[idx_v.at[pl.ds(c * C, C)]], rows_v.at[c], gsem.at[c]
            )
            for c in range(NCH)
        ]
        writes = []
        for c in range(NCH):
            gathers[c].wait()
            writes.append(
                pltpu.async_copy(
                    rows_v.at[c], out_hbm.at[pl.ds(base + c * C, C)], wsem
                )
            )
        for w in writes:
            w.wait()

    return gather_kernel


@jax.jit
def kernel(label_ids, table):
    B = label_ids.shape[0]
    V, D = table.shape
    NCH = 4
    out = _make_gather(V, D, B, NCH)(table, label_ids.astype(jnp.int32))
    return out[:, None, :]


# speedup vs baseline: 1.4672x; 1.4616x over previous
import functools

import jax
import jax.numpy as jnp
from jax import lax
from jax.experimental import pallas as pl
from jax.experimental.pallas import tpu as pltpu
from jax.experimental.pallas import tpu_sc as plsc


@functools.lru_cache(maxsize=None)
def _make_gather(V, D, B):
    info = plsc.get_sparse_core_info()
    NC, NS, L = info.num_cores, info.num_subcores, info.num_lanes
    NW = NC * NS
    assert B % (8 * NW) == 0 and D % L == 0
    b_per_w = B // NW
    CB = 128
    NCHK = b_per_w // CB
    mesh = plsc.VectorSubcoreMesh(core_axis_name="c", subcore_axis_name="s")

    @functools.partial(
        pl.kernel,
        mesh=mesh,
        out_type=jax.ShapeDtypeStruct((B, D), jnp.float32),
        scratch_types=[
            pltpu.VMEM((b_per_w,), jnp.int32),
            pltpu.VMEM((b_per_w, D), jnp.float32),
            pltpu.SemaphoreType.DMA((NCHK,)),
            pltpu.SemaphoreType.DMA,
        ],
        compiler_params=pltpu.CompilerParams(use_tc_tiling_on_sc=True),
    )
    def gather_kernel(table_hbm, idx_hbm, out_hbm, idx_v, rows_v, gsem, wsem):
        wid = lax.axis_index("s") * NC + lax.axis_index("c")
        base = wid * b_per_w
        pltpu.sync_copy(idx_hbm.at[pl.ds(base, b_per_w)], idx_v)

        def fire(c):
            def body(g, _):
                v = idx_v[pl.ds(g * L, L)]
                for q in range(L):
                    pltpu.async_copy(
                        table_hbm.at[pl.ds(v[q], 1)],
                        rows_v.at[pl.ds(g * L + q, 1)],
                        gsem.at[c],
                    )
                return 0

            lax.fori_loop(c * (CB // L), (c + 1) * (CB // L), body, 0)

        def drain(c):
            pltpu.make_async_copy(
                table_hbm.at[pl.ds(0, CB)],
                rows_v.at[pl.ds(c * CB, CB)],
                gsem.at[c],
            ).wait()

        fire(0)
        if NCHK > 1:
            fire(1)
        writes = []
        for c in range(NCHK):
            drain(c)
            writes.append(
                pltpu.async_copy(
                    rows_v.at[pl.ds(c * CB, CB)],
                    out_hbm.at[pl.ds(base + c * CB, CB)],
                    wsem,
                )
            )
            if c + 2 < NCHK:
                fire(c + 2)
        for w in writes:
            w.wait()

    return gather_kernel


@jax.jit
def kernel(label_ids, table):
    B = label_ids.shape[0]
    V, D = table.shape
    out = _make_gather(V, D, B)(table, label_ids.astype(jnp.int32))
    return out[:, None, :]


# speedup vs baseline: 1.4789x; 1.0080x over previous
import functools

import jax
import jax.numpy as jnp
from jax import lax
from jax.experimental import pallas as pl
from jax.experimental.pallas import tpu as pltpu
from jax.experimental.pallas import tpu_sc as plsc


@functools.lru_cache(maxsize=None)
def _make_gather(V, D, B):
    info = plsc.get_sparse_core_info()
    NC, NS, L = info.num_cores, info.num_subcores, info.num_lanes
    NW = NC * NS
    assert B % (8 * NW) == 0 and D % L == 0
    b_per_w = B // NW
    CB = 128
    NCHK = b_per_w // CB
    mesh = plsc.VectorSubcoreMesh(core_axis_name="c", subcore_axis_name="s")

    @functools.partial(
        pl.kernel,
        mesh=mesh,
        out_type=jax.ShapeDtypeStruct((B, D), jnp.float32),
        scratch_types=[
            pltpu.VMEM((b_per_w,), jnp.int32),
            pltpu.VMEM((b_per_w, D), jnp.float32),
            pltpu.SemaphoreType.DMA((NCHK,)),
            pltpu.SemaphoreType.DMA,
        ],
        compiler_params=pltpu.CompilerParams(use_tc_tiling_on_sc=True),
    )
    def gather_kernel(table_hbm, idx_hbm, out_hbm, idx_v, rows_v, gsem, wsem):
        wid = lax.axis_index("s") * NC + lax.axis_index("c")
        base = wid * b_per_w
        pltpu.sync_copy(idx_hbm.at[pl.ds(base, b_per_w)], idx_v)

        def fire(c):
            def body(g, _):
                v = idx_v[pl.ds(g * L, L)]
                for q in range(L):
                    pltpu.async_copy(
                        table_hbm.at[pl.ds(v[q], 1)],
                        rows_v.at[pl.ds(g * L + q, 1)],
                        gsem.at[c],
                    )
                return 0

            lax.fori_loop(c * (CB // L), (c + 1) * (CB // L), body, 0)

        def drain(c):
            pltpu.make_async_copy(
                table_hbm.at[pl.ds(0, CB)],
                rows_v.at[pl.ds(c * CB, CB)],
                gsem.at[c],
            ).wait()

        for c in range(NCHK):
            fire(c)
        writes = []
        for c in range(NCHK):
            drain(c)
            writes.append(
                pltpu.async_copy(
                    rows_v.at[pl.ds(c * CB, CB)],
                    out_hbm.at[pl.ds(base + c * CB, CB)],
                    wsem,
                )
            )
        for w in writes:
            w.wait()

    return gather_kernel


@jax.jit
def kernel(label_ids, table):
    B = label_ids.shape[0]
    V, D = table.shape
    out = _make_gather(V, D, B)(table, label_ids.astype(jnp.int32))
    return out[:, None, :]
